# baseline (speedup 1.0000x reference)
import functools

import jax
import jax.numpy as jnp
from jax import lax
from jax.experimental import pallas as pl
from jax.experimental.pallas import tpu as pltpu
from jax.experimental.pallas import tpu_sc as plsc

_LANE = 128
_NT = 16
_NC = 2
_CHUNK = 128


def _agg(x_cb, sd, n_pad):
    ncb, n, lane = x_cb.shape
    nch = sd.shape[1]
    npass = ncb // _NC
    rpt = (n // _NT) & ~7
    tail0 = rpt * _NT
    tail = n - tail0
    mesh = plsc.VectorSubcoreMesh(core_axis_name="c", subcore_axis_name="s")

    assert nch % 6 == 0 and nch >= 12
    nq = nch // 6

    @functools.partial(
        pl.kernel,
        mesh=mesh,
        out_type=jax.ShapeDtypeStruct((ncb, n, lane), jnp.float32),
        scratch_types=[
            [pltpu.VMEM((2, _CHUNK), jnp.int32) for _ in range(6)],
            [pltpu.VMEM((_CHUNK, lane), jnp.float32) for _ in range(3)],
            pltpu.VMEM_SHARED((n_pad, lane), jnp.float32),
            [pltpu.SemaphoreType.DMA for _ in range(6)],
            [pltpu.SemaphoreType.DMA for _ in range(3)],
            [pltpu.SemaphoreType.DMA for _ in range(3)],
        ],
    )
    def body(x_hbm, sd_hbm, out_hbm, idxb, rows, acc_sh, isem, gsem, ssem):
        cid = lax.axis_index("c")
        sid = lax.axis_index("s")
        r0 = sid * rpt

        for p in range(npass):
            blk = cid * npass + p

            def i_start(t, ib):
                pltpu.async_copy(sd_hbm.at[sid, t], idxb[ib], isem[ib])

            def i_wait(t, ib):
                pltpu.make_async_copy(sd_hbm.at[sid, t], idxb[ib],
                                      isem[ib]).wait()

            def g_start(t, b, ib):
                pltpu.async_copy(x_hbm.at[blk].at[idxb[ib].at[0]], rows[b],
                                 gsem[b])

            def g_wait(t, b, ib):
                pltpu.make_async_copy(x_hbm.at[blk].at[idxb[ib].at[0]],
                                      rows[b], gsem[b]).wait()

            def s_start(t, b, ib):
                pltpu.async_copy(rows[b], acc_sh.at[idxb[ib].at[1]], ssem[b],
                                 add=True)

            def s_wait(t, b, ib):
                pltpu.make_async_copy(rows[b], acc_sh.at[idxb[ib].at[1]],
                                      ssem[b]).wait()

            def chunk(t, j):
                s_wait(t - 3, j % 3, (j + 3) % 6)
                i_start(t + 3, (j + 3) % 6)
                i_wait(t, j)
                g_start(t, j % 3, j)
                g_wait(t - 1, (j + 2) % 3, (j + 5) % 6)
                s_start(t - 1, (j + 2) % 3, (j + 5) % 6)

            pltpu.sync_copy(x_hbm.at[blk, pl.ds(r0, rpt)], acc_sh.at[pl.ds(r0, rpt)])
            if tail:
                @pl.when(sid == 0)
                def _():
                    pltpu.sync_copy(x_hbm.at[blk, pl.ds(tail0, tail)],
                                    acc_sh.at[pl.ds(tail0, tail)])
            plsc.subcore_barrier()

            i_start(0, 0)
            i_start(1, 1)
            i_start(2, 2)
            i_wait(0, 0)
            g_start(0, 0, 0)
            i_start(3, 3)
            i_wait(1, 1)
            g_start(1, 1, 1)
            g_wait(0, 0, 0)
            s_start(0, 0, 0)
            i_start(4, 4)
            i_wait(2, 2)
            g_start(2, 2, 2)
            g_wait(1, 1, 1)
            s_start(1, 1, 1)
            i_start(5, 5)
            chunk(3, 3)
            chunk(4, 4)
            chunk(5, 5)

            def hexa(q, carry):
                t0 = q * 6
                for j in range(6):
                    chunk(t0 + j, j)
                return carry

            lax.fori_loop(1, nq - 1, hexa, 0)

            t0 = nch - 6
            for j in range(6):
                t = t0 + j
                s_wait(t - 3, j % 3, (j + 3) % 6)
                if j < 3:
                    i_start(t + 3, (j + 3) % 6)
                i_wait(t, j)
                g_start(t, j % 3, j)
                g_wait(t - 1, (j + 2) % 3, (j + 5) % 6)
                s_start(t - 1, (j + 2) % 3, (j + 5) % 6)
            g_wait(nch - 1, 2, 5)
            s_start(nch - 1, 2, 5)
            s_wait(nch - 3, 0, 3)
            s_wait(nch - 2, 1, 4)
            s_wait(nch - 1, 2, 5)

            plsc.subcore_barrier()
            pltpu.sync_copy(acc_sh.at[pl.ds(r0, rpt)], out_hbm.at[blk, pl.ds(r0, rpt)])
            if tail:
                @pl.when(sid == 0)
                def _():
                    pltpu.sync_copy(acc_sh.at[pl.ds(tail0, tail)],
                                    out_hbm.at[blk, pl.ds(tail0, tail)])
            if p + 1 < npass:
                plsc.subcore_barrier()

    return body(x_cb, sd)


def _mlp_layer(h_cb, W1b, b1, g, be, W2, b2, block_rows):
    ncb_in, n, _ = h_cb.shape
    H = W2.shape[0]
    ncb_out = H // _LANE
    nb = n // block_rows

    def body(h_ref, W1_ref, b1_ref, g_ref, be_ref, W2_ref, b2_ref, out_ref,
             sum_ref, sq_ref):
        p = pl.program_id(0)
        i = pl.program_id(1)
        h1 = b1_ref[...]
        for c in range(ncb_in):
            h1 = h1 + jnp.dot(h_ref[c], W1_ref[c],
                              preferred_element_type=jnp.float32)

        @pl.when(jnp.logical_and(p == 0, i == 0))
        def _():
            sum_ref[...] = jnp.zeros_like(sum_ref)
            sq_ref[...] = jnp.zeros_like(sq_ref)

        @pl.when(p == 0)
        def _():
            sum_ref[...] += jnp.sum(h1, axis=0, keepdims=True)
            sq_ref[...] += jnp.sum(h1 * h1, axis=0, keepdims=True)

        @pl.when(p == 1)
        def _():
            inv_n = 1.0 / n
            mu = sum_ref[...] * inv_n
            var = sq_ref[...] * inv_n - mu * mu
            scale = g_ref[...] * lax.rsqrt(var + 1e-5)
            shift = be_ref[...] - mu * scale
            r = jnp.maximum(h1 * scale + shift, 0.0)
            h2 = jnp.dot(r, W2_ref[...], preferred_element_type=jnp.float32)
            y = jnp.maximum(h2 + b2_ref[...], 0.0)
            for c in range(ncb_out):
                out_ref[c] = y[:, c * _LANE:(c + 1) * _LANE]

    return pl.pallas_call(
        body,
        grid=(2, nb),
        in_specs=[
            pl.BlockSpec((ncb_in, block_rows, _LANE), lambda p, i: (0, i, 0)),
            pl.BlockSpec((ncb_in, _LANE, H), lambda p, i: (0, 0, 0)),
            pl.BlockSpec((1, H), lambda p, i: (0, 0)),
            pl.BlockSpec((1, H), lambda p, i: (0, 0)),
            pl.BlockSpec((1, H), lambda p, i: (0, 0)),
            pl.BlockSpec((H, H), lambda p, i: (0, 0)),
            pl.BlockSpec((1, H), lambda p, i: (0, 0)),
        ],
        out_specs=pl.BlockSpec((ncb_out, block_rows, _LANE), lambda p, i: (0, i, 0)),
        out_shape=jax.ShapeDtypeStruct((ncb_out, n, _LANE), jnp.float32),
        scratch_shapes=[
            pltpu.VMEM((1, H), jnp.float32),
            pltpu.VMEM((1, H), jnp.float32),
        ],
    )(h_cb, W1b, b1, g, be, W2, b2)


def _mlp_final(h_cb, batch_r, W1b, b1, g, be, W2, b2, Wf1, bf1, Wf2, bf2,
               n_graphs, block_rows):
    ncb_in, n, _ = h_cb.shape
    H = W2.shape[0]
    OUT = Wf2.shape[1]
    nb = n // block_rows

    def body(h_ref, batch_ref, W1_ref, b1_ref, g_ref, be_ref, W2_ref, b2_ref,
             Wf1_ref, bf1_ref, Wf2_ref, bf2_ref, out_ref, sum_ref, sq_ref,
             pool_ref):
        p = pl.program_id(0)
        i = pl.program_id(1)
        h1 = b1_ref[...]
        for c in range(ncb_in):
            h1 = h1 + jnp.dot(h_ref[c], W1_ref[c],
                              preferred_element_type=jnp.float32)

        @pl.when(jnp.logical_and(p == 0, i == 0))
        def _():
            sum_ref[...] = jnp.zeros_like(sum_ref)
            sq_ref[...] = jnp.zeros_like(sq_ref)
            pool_ref[...] = jnp.zeros_like(pool_ref)

        @pl.when(p == 0)
        def _():
            sum_ref[...] += jnp.sum(h1, axis=0, keepdims=True)
            sq_ref[...] += jnp.sum(h1 * h1, axis=0, keepdims=True)

        @pl.when(p == 1)
        def _():
            inv_n = 1.0 / n
            mu = sum_ref[...] * inv_n
            var = sq_ref[...] * inv_n - mu * mu
            scale = g_ref[...] * lax.rsqrt(var + 1e-5)
            shift = be_ref[...] - mu * scale
            r = jnp.maximum(h1 * scale + shift, 0.0)
            h2 = jnp.dot(r, W2_ref[...], preferred_element_type=jnp.float32)
            y = jnp.maximum(h2 + b2_ref[...], 0.0)
            ids = batch_ref[0, 0, :]
            oh = (ids[:, None] == lax.broadcasted_iota(
                jnp.int32, (1, n_graphs), 1)).astype(jnp.float32)
            pool_ref[...] += lax.dot_general(
                oh, y, (((0,), (0,)), ((), ())),
                preferred_element_type=jnp.float32)

        @pl.when(jnp.logical_and(p == 1, i == nb - 1))
        def _():
            pooled = pool_ref[...]
            hf = jnp.maximum(
                jnp.dot(pooled, Wf1_ref[...], preferred_element_type=jnp.float32)
                + bf1_ref[...], 0.0)
            out_ref[...] = (jnp.dot(hf, Wf2_ref[...],
                                    preferred_element_type=jnp.float32)
                            + bf2_ref[...])

    return pl.pallas_call(
        body,
        grid=(2, nb),
        in_specs=[
            pl.BlockSpec((ncb_in, block_rows, _LANE), lambda p, i: (0, i, 0)),
            pl.BlockSpec((1, 1, block_rows), lambda p, i: (i, 0, 0)),
            pl.BlockSpec((ncb_in, _LANE, H), lambda p, i: (0, 0, 0)),
            pl.BlockSpec((1, H), lambda p, i: (0, 0)),
            pl.BlockSpec((1, H), lambda p, i: (0, 0)),
            pl.BlockSpec((1, H), lambda p, i: (0, 0)),
            pl.BlockSpec((H, H), lambda p, i: (0, 0)),
            pl.BlockSpec((1, H), lambda p, i: (0, 0)),
            pl.BlockSpec((H, H), lambda p, i: (0, 0)),
            pl.BlockSpec((1, H), lambda p, i: (0, 0)),
            pl.BlockSpec((H, OUT), lambda p, i: (0, 0)),
            pl.BlockSpec((1, OUT), lambda p, i: (0, 0)),
        ],
        out_specs=pl.BlockSpec((n_graphs, OUT), lambda p, i: (0, 0)),
        out_shape=jax.ShapeDtypeStruct((n_graphs, OUT), jnp.float32),
        scratch_shapes=[
            pltpu.VMEM((1, H), jnp.float32),
            pltpu.VMEM((1, H), jnp.float32),
            pltpu.VMEM((n_graphs, H), jnp.float32),
        ],
    )(h_cb, batch_r, W1b, b1, g, be, W2, b2, Wf1, bf1, Wf2, bf2)


def kernel(x, edge_index, batch,
           W1_0, b1_0, g_0, be_0, W2_0, b2_0,
           W1_1, b1_1, g_1, be_1, W2_1, b2_1,
           W1_2, b1_2, g_2, be_2, W2_2, b2_2,
           Wf1, bf1, Wf2, bf2):
    n, din = x.shape
    e = edge_index.shape[1]
    H = W2_0.shape[0]
    G = 128
    n_pad = n + 8
    block_rows = 1000

    x_cb = jnp.transpose(x.reshape(n, din // _LANE, _LANE), (1, 0, 2))
    nch = (-(-e // _NT) + _CHUNK - 1) // _CHUNK
    nch = -(-nch // 6) * 6
    e_pad = _NT * nch * _CHUNK
    src = jnp.concatenate(
        [edge_index[0], jnp.zeros((e_pad - e,), dtype=jnp.int32)])
    pad_dst = n + (jnp.arange(e_pad - e, dtype=jnp.int32) % 8)
    dst = jnp.concatenate([edge_index[1], pad_dst])
    sd = jnp.stack([src.reshape(_NT, nch, _CHUNK),
                    dst.reshape(_NT, nch, _CHUNK)], axis=2)
    batch_r = batch.reshape(n // block_rows, 1, block_rows)

    def wblocks(W):
        return W.reshape(W.shape[0] // _LANE, _LANE, W.shape[1])

    def rowvec(v):
        return v.reshape(1, v.shape[0])

    h = _agg(x_cb, sd, n_pad)
    x1 = _mlp_layer(h, wblocks(W1_0), rowvec(b1_0), rowvec(g_0), rowvec(be_0),
                    W2_0, rowvec(b2_0), block_rows)
    h = _agg(x1, sd, n_pad)
    x2 = _mlp_layer(h, wblocks(W1_1), rowvec(b1_1), rowvec(g_1), rowvec(be_1),
                    W2_1, rowvec(b2_1), block_rows)
    h = _agg(x2, sd, n_pad)
    out = _mlp_final(h, batch_r, wblocks(W1_2), rowvec(b1_2), rowvec(g_2),
                     rowvec(be_2), W2_2, rowvec(b2_2), Wf1, rowvec(bf1),
                     Wf2, rowvec(bf2), G, block_rows)
    return out


# speedup vs baseline: 2.4738x; 2.4738x over previous
import functools

import jax
import jax.numpy as jnp
from jax import lax
from jax.experimental import pallas as pl
from jax.experimental.pallas import tpu as pltpu
from jax.experimental.pallas import tpu_sc as plsc

_LANE = 128
_NT = 16
_NC = 2
_CHUNK = 128


def _agg(x_cb, sd, n_pad):
    ncb, n, lane = x_cb.shape
    nch = sd.shape[1]
    npass = ncb // _NC
    rpt = (n // _NT) & ~7
    tail0 = rpt * _NT
    tail = n - tail0
    mesh = plsc.VectorSubcoreMesh(core_axis_name="c", subcore_axis_name="s")

    assert nch % 4 == 0 and nch >= 8
    nq = nch // 4

    @functools.partial(
        pl.kernel,
        mesh=mesh,
        out_type=jax.ShapeDtypeStruct((ncb, n, lane), jnp.float32),
        scratch_types=[
            [pltpu.VMEM((2, _CHUNK), jnp.int32) for _ in range(4)],
            [pltpu.VMEM((_CHUNK, lane), jnp.float32) for _ in range(2)],
            pltpu.VMEM_SHARED((n_pad, lane), jnp.float32),
            [pltpu.SemaphoreType.DMA for _ in range(4)],
            [pltpu.SemaphoreType.DMA for _ in range(2)],
            [pltpu.SemaphoreType.DMA for _ in range(2)],
        ],
    )
    def body(x_hbm, sd_hbm, out_hbm, idxb, rows, acc_sh, isem, gsem, ssem):
        cid = lax.axis_index("c")
        sid = lax.axis_index("s")
        r0 = sid * rpt

        for p in range(npass):
            blk = cid * npass + p

            def i_start(t, ib):
                pltpu.async_copy(sd_hbm.at[sid, t], idxb[ib], isem[ib])

            def i_wait(t, ib):
                pltpu.make_async_copy(sd_hbm.at[sid, t], idxb[ib],
                                      isem[ib]).wait()

            def g_start(t, b, ib):
                pltpu.async_copy(x_hbm.at[blk].at[idxb[ib].at[0]], rows[b],
                                 gsem[b])

            def g_wait(t, b, ib):
                pltpu.make_async_copy(x_hbm.at[blk].at[idxb[ib].at[0]],
                                      rows[b], gsem[b]).wait()

            def s_start(t, b, ib):
                pltpu.async_copy(rows[b], acc_sh.at[idxb[ib].at[1]], ssem[b],
                                 add=True)

            def s_wait(t, b, ib):
                pltpu.make_async_copy(rows[b], acc_sh.at[idxb[ib].at[1]],
                                      ssem[b]).wait()

            def chunk(t, j):
                s_wait(t - 2, j % 2, (j + 2) % 4)
                i_start(t + 2, (j + 2) % 4)
                i_wait(t, j)
                g_start(t, j % 2, j)
                g_wait(t - 1, (j + 1) % 2, (j + 3) % 4)
                s_start(t - 1, (j + 1) % 2, (j + 3) % 4)

            pltpu.sync_copy(x_hbm.at[blk, pl.ds(r0, rpt)], acc_sh.at[pl.ds(r0, rpt)])
            if tail:
                @pl.when(sid == 0)
                def _():
                    pltpu.sync_copy(x_hbm.at[blk, pl.ds(tail0, tail)],
                                    acc_sh.at[pl.ds(tail0, tail)])
            plsc.subcore_barrier()

            i_start(0, 0)
            i_start(1, 1)
            i_start(2, 2)
            i_wait(0, 0)
            g_start(0, 0, 0)
            i_start(3, 3)
            i_wait(1, 1)
            g_start(1, 1, 1)
            g_wait(0, 0, 0)
            s_start(0, 0, 0)
            chunk(2, 2)
            chunk(3, 3)

            def quad(q, carry):
                t0 = q * 4
                for j in range(4):
                    chunk(t0 + j, j)
                return carry

            lax.fori_loop(1, nq - 1, quad, 0)

            t0 = nch - 4
            for j in range(4):
                t = t0 + j
                s_wait(t - 2, j % 2, (j + 2) % 4)
                if j < 2:
                    i_start(t + 2, (j + 2) % 4)
                i_wait(t, j)
                g_start(t, j % 2, j)
                g_wait(t - 1, (j + 1) % 2, (j + 3) % 4)
                s_start(t - 1, (j + 1) % 2, (j + 3) % 4)
            g_wait(nch - 1, (nch - 1) % 2, 3)
            s_start(nch - 1, (nch - 1) % 2, 3)
            s_wait(nch - 2, (nch - 2) % 2, 2)
            s_wait(nch - 1, (nch - 1) % 2, 3)

            plsc.subcore_barrier()
            pltpu.sync_copy(acc_sh.at[pl.ds(r0, rpt)], out_hbm.at[blk, pl.ds(r0, rpt)])
            if tail:
                @pl.when(sid == 0)
                def _():
                    pltpu.sync_copy(acc_sh.at[pl.ds(tail0, tail)],
                                    out_hbm.at[blk, pl.ds(tail0, tail)])
            if p + 1 < npass:
                plsc.subcore_barrier()

    return body(x_cb, sd)


def _mlp_layer(h_cb, W1b, b1, g, be, W2, b2, block_rows):
    ncb_in, n, _ = h_cb.shape
    H = W2.shape[0]
    ncb_out = H // _LANE
    nb = n // block_rows

    def body(h_ref, W1_ref, b1_ref, g_ref, be_ref, W2_ref, b2_ref, out_ref,
             sum_ref, sq_ref):
        p = pl.program_id(0)
        i = pl.program_id(1)
        h1 = b1_ref[...]
        for c in range(ncb_in):
            h1 = h1 + jnp.dot(h_ref[c], W1_ref[c],
                              preferred_element_type=jnp.float32)

        @pl.when(jnp.logical_and(p == 0, i == 0))
        def _():
            sum_ref[...] = jnp.zeros_like(sum_ref)
            sq_ref[...] = jnp.zeros_like(sq_ref)

        @pl.when(p == 0)
        def _():
            sum_ref[...] += jnp.sum(h1, axis=0, keepdims=True)
            sq_ref[...] += jnp.sum(h1 * h1, axis=0, keepdims=True)

        @pl.when(p == 1)
        def _():
            inv_n = 1.0 / n
            mu = sum_ref[...] * inv_n
            var = sq_ref[...] * inv_n - mu * mu
            scale = g_ref[...] * lax.rsqrt(var + 1e-5)
            shift = be_ref[...] - mu * scale
            r = jnp.maximum(h1 * scale + shift, 0.0)
            h2 = jnp.dot(r, W2_ref[...], preferred_element_type=jnp.float32)
            y = jnp.maximum(h2 + b2_ref[...], 0.0)
            for c in range(ncb_out):
                out_ref[c] = y[:, c * _LANE:(c + 1) * _LANE]

    return pl.pallas_call(
        body,
        grid=(2, nb),
        in_specs=[
            pl.BlockSpec((ncb_in, block_rows, _LANE), lambda p, i: (0, i, 0)),
            pl.BlockSpec((ncb_in, _LANE, H), lambda p, i: (0, 0, 0)),
            pl.BlockSpec((1, H), lambda p, i: (0, 0)),
            pl.BlockSpec((1, H), lambda p, i: (0, 0)),
            pl.BlockSpec((1, H), lambda p, i: (0, 0)),
            pl.BlockSpec((H, H), lambda p, i: (0, 0)),
            pl.BlockSpec((1, H), lambda p, i: (0, 0)),
        ],
        out_specs=pl.BlockSpec((ncb_out, block_rows, _LANE), lambda p, i: (0, i, 0)),
        out_shape=jax.ShapeDtypeStruct((ncb_out, n, _LANE), jnp.float32),
        scratch_shapes=[
            pltpu.VMEM((1, H), jnp.float32),
            pltpu.VMEM((1, H), jnp.float32),
        ],
    )(h_cb, W1b, b1, g, be, W2, b2)


def _mlp_final(h_cb, batch_r, W1b, b1, g, be, W2, b2, Wf1, bf1, Wf2, bf2,
               n_graphs, block_rows):
    ncb_in, n, _ = h_cb.shape
    H = W2.shape[0]
    OUT = Wf2.shape[1]
    nb = n // block_rows

    def body(h_ref, batch_ref, W1_ref, b1_ref, g_ref, be_ref, W2_ref, b2_ref,
             Wf1_ref, bf1_ref, Wf2_ref, bf2_ref, out_ref, sum_ref, sq_ref,
             pool_ref):
        p = pl.program_id(0)
        i = pl.program_id(1)
        h1 = b1_ref[...]
        for c in range(ncb_in):
            h1 = h1 + jnp.dot(h_ref[c], W1_ref[c],
                              preferred_element_type=jnp.float32)

        @pl.when(jnp.logical_and(p == 0, i == 0))
        def _():
            sum_ref[...] = jnp.zeros_like(sum_ref)
            sq_ref[...] = jnp.zeros_like(sq_ref)
            pool_ref[...] = jnp.zeros_like(pool_ref)

        @pl.when(p == 0)
        def _():
            sum_ref[...] += jnp.sum(h1, axis=0, keepdims=True)
            sq_ref[...] += jnp.sum(h1 * h1, axis=0, keepdims=True)

        @pl.when(p == 1)
        def _():
            inv_n = 1.0 / n
            mu = sum_ref[...] * inv_n
            var = sq_ref[...] * inv_n - mu * mu
            scale = g_ref[...] * lax.rsqrt(var + 1e-5)
            shift = be_ref[...] - mu * scale
            r = jnp.maximum(h1 * scale + shift, 0.0)
            h2 = jnp.dot(r, W2_ref[...], preferred_element_type=jnp.float32)
            y = jnp.maximum(h2 + b2_ref[...], 0.0)
            ids = batch_ref[0, 0, :]
            oh = (ids[:, None] == lax.broadcasted_iota(
                jnp.int32, (1, n_graphs), 1)).astype(jnp.float32)
            pool_ref[...] += lax.dot_general(
                oh, y, (((0,), (0,)), ((), ())),
                preferred_element_type=jnp.float32)

        @pl.when(jnp.logical_and(p == 1, i == nb - 1))
        def _():
            pooled = pool_ref[...]
            hf = jnp.maximum(
                jnp.dot(pooled, Wf1_ref[...], preferred_element_type=jnp.float32)
                + bf1_ref[...], 0.0)
            out_ref[...] = (jnp.dot(hf, Wf2_ref[...],
                                    preferred_element_type=jnp.float32)
                            + bf2_ref[...])

    return pl.pallas_call(
        body,
        grid=(2, nb),
        in_specs=[
            pl.BlockSpec((ncb_in, block_rows, _LANE), lambda p, i: (0, i, 0)),
            pl.BlockSpec((1, 1, block_rows), lambda p, i: (i, 0, 0)),
            pl.BlockSpec((ncb_in, _LANE, H), lambda p, i: (0, 0, 0)),
            pl.BlockSpec((1, H), lambda p, i: (0, 0)),
            pl.BlockSpec((1, H), lambda p, i: (0, 0)),
            pl.BlockSpec((1, H), lambda p, i: (0, 0)),
            pl.BlockSpec((H, H), lambda p, i: (0, 0)),
            pl.BlockSpec((1, H), lambda p, i: (0, 0)),
            pl.BlockSpec((H, H), lambda p, i: (0, 0)),
            pl.BlockSpec((1, H), lambda p, i: (0, 0)),
            pl.BlockSpec((H, OUT), lambda p, i: (0, 0)),
            pl.BlockSpec((1, OUT), lambda p, i: (0, 0)),
        ],
        out_specs=pl.BlockSpec((n_graphs, OUT), lambda p, i: (0, 0)),
        out_shape=jax.ShapeDtypeStruct((n_graphs, OUT), jnp.float32),
        scratch_shapes=[
            pltpu.VMEM((1, H), jnp.float32),
            pltpu.VMEM((1, H), jnp.float32),
            pltpu.VMEM((n_graphs, H), jnp.float32),
        ],
    )(h_cb, batch_r, W1b, b1, g, be, W2, b2, Wf1, bf1, Wf2, bf2)


def kernel(x, edge_index, batch,
           W1_0, b1_0, g_0, be_0, W2_0, b2_0,
           W1_1, b1_1, g_1, be_1, W2_1, b2_1,
           W1_2, b1_2, g_2, be_2, W2_2, b2_2,
           Wf1, bf1, Wf2, bf2):
    n, din = x.shape
    e = edge_index.shape[1]
    H = W2_0.shape[0]
    G = 128
    n_pad = n + 8
    block_rows = 1000

    x_cb = jnp.transpose(x.reshape(n, din // _LANE, _LANE), (1, 0, 2))
    nch = (-(-e // _NT) + _CHUNK - 1) // _CHUNK
    nch = -(-nch // 4) * 4
    e_pad = _NT * nch * _CHUNK
    src = jnp.concatenate(
        [edge_index[0], jnp.zeros((e_pad - e,), dtype=jnp.int32)])
    pad_dst = n + (jnp.arange(e_pad - e, dtype=jnp.int32) % 8)
    dst = jnp.concatenate([edge_index[1], pad_dst])
    sd = jnp.stack([src.reshape(_NT, nch, _CHUNK),
                    dst.reshape(_NT, nch, _CHUNK)], axis=2)
    batch_r = batch.reshape(n // block_rows, 1, block_rows)

    def wblocks(W):
        return W.reshape(W.shape[0] // _LANE, _LANE, W.shape[1])

    def rowvec(v):
        return v.reshape(1, v.shape[0])

    h = _agg(x_cb, sd, n_pad)
    x1 = _mlp_layer(h, wblocks(W1_0), rowvec(b1_0), rowvec(g_0), rowvec(be_0),
                    W2_0, rowvec(b2_0), block_rows)
    h = _agg(x1, sd, n_pad)
    x2 = _mlp_layer(h, wblocks(W1_1), rowvec(b1_1), rowvec(g_1), rowvec(be_1),
                    W2_1, rowvec(b2_1), block_rows)
    h = _agg(x2, sd, n_pad)
    out = _mlp_final(h, batch_r, wblocks(W1_2), rowvec(b1_2), rowvec(g_2),
                     rowvec(be_2), W2_2, rowvec(b2_2), Wf1, rowvec(bf1),
                     Wf2, rowvec(bf2), G, block_rows)
    return out
